# Initial kernel scaffold; baseline (speedup 1.0000x reference)
import jax
import jax.numpy as jnp
from jax.experimental import pallas as pl


def kernel(x, edge_index, W1, b1, W2, b2, Wc, bc):
    raise NotImplementedError("write your pallas kernel here")


# speedup vs baseline: 13.4494x; 13.4494x over previous
import functools

import jax
import jax.numpy as jnp
from jax import lax
from jax.experimental import pallas as pl
from jax.experimental.pallas import tpu as pltpu
import jax.experimental.pallas.tpu_sc as plsc

_NC = 2
_NS = 16
_NW = _NC * _NS
_L = 16

_N = 10000
_E = 320000
_D = 128

_CH = 80
_CHUNKS = _E // _CH
_CPW = _CHUNKS // _NW

_NP = 10240
_SEG = _NP // _NS

_NR = 10240
_RPT = _NR // _NS
_RCH = 128

_BR = 1000


def _sc_mesh():
  return plsc.VectorSubcoreMesh(
      core_axis_name="c", subcore_axis_name="s",
      num_cores=_NC, num_subcores=_NS)


def _hist_body(dst_hbm, out_hbm, didx_v, ones_v, zbuf_v, acc_sh):
  cid = lax.axis_index("c")
  sid = lax.axis_index("s")
  wid = sid * _NC + cid

  for j in range(_CH // _L):
    ones_v[pl.ds(j * _L, _L)] = jnp.ones((_L,), jnp.float32)
  for j in range(_SEG // _L):
    zbuf_v[pl.ds(j * _L, _L)] = jnp.zeros((_L,), jnp.float32)
  pltpu.sync_copy(zbuf_v, acc_sh.at[pl.ds(sid * _SEG, _SEG)])
  plsc.subcore_barrier()

  base = wid * _CPW * _CH

  def chunk(c, carry):
    pltpu.sync_copy(dst_hbm.at[pl.ds(base + c * _CH, _CH)], didx_v)
    pltpu.sync_copy(ones_v, acc_sh.at[didx_v], add=True)
    return carry

  lax.fori_loop(0, _CPW, chunk, 0)
  plsc.subcore_barrier()

  pltpu.sync_copy(acc_sh.at[pl.ds(sid * _SEG, _SEG)], zbuf_v)
  pltpu.sync_copy(zbuf_v, out_hbm.at[cid, pl.ds(sid * _SEG, _SEG)])


_hist_call = functools.partial(
    pl.kernel,
    out_type=jax.ShapeDtypeStruct((_NC, _NP), jnp.float32),
    mesh=_sc_mesh(),
    scratch_types=[
        pltpu.VMEM((_CH,), jnp.int32),
        pltpu.VMEM((_CH,), jnp.float32),
        pltpu.VMEM((_SEG,), jnp.float32),
        pltpu.VMEM_SHARED((_NP,), jnp.float32),
    ],
)


def _agg_body(p_hbm, src_hbm, dst_hbm, out_hbm,
              sidx_v, didx_v, rows_v, bnc_v, acc_sh, sem):
  cid = lax.axis_index("c")
  sid = lax.axis_index("s")
  wid = sid * _NC + cid

  def zrow(r, carry):
    for j in range(_D // _L):
      bnc_v[r, pl.ds(j * _L, _L)] = jnp.zeros((_L,), jnp.float32)
    return carry

  lax.fori_loop(0, _RCH, zrow, 0)
  for t in range(_RPT // _RCH):
    pltpu.sync_copy(bnc_v, acc_sh.at[pl.ds(sid * _RPT + t * _RCH, _RCH)])
  plsc.subcore_barrier()

  base = wid * _CPW * _CH

  def chunk(c, carry):
    pltpu.sync_copy(src_hbm.at[pl.ds(base + c * _CH, _CH)], sidx_v)
    pltpu.sync_copy(dst_hbm.at[pl.ds(base + c * _CH, _CH)], didx_v)
    pltpu.async_copy(p_hbm.at[sidx_v], rows_v, sem).wait()
    pltpu.sync_copy(rows_v, acc_sh.at[didx_v], add=True)
    return carry

  lax.fori_loop(0, _CPW, chunk, 0)
  plsc.subcore_barrier()

  for t in range(_RPT // _RCH):
    pltpu.sync_copy(acc_sh.at[pl.ds(sid * _RPT + t * _RCH, _RCH)], bnc_v)
    pltpu.sync_copy(bnc_v, out_hbm.at[cid, pl.ds(sid * _RPT + t * _RCH, _RCH)])


_agg_call = functools.partial(
    pl.kernel,
    out_type=jax.ShapeDtypeStruct((_NC, _NR, _D), jnp.float32),
    mesh=_sc_mesh(),
    scratch_types=[
        pltpu.VMEM((_CH,), jnp.int32),
        pltpu.VMEM((_CH,), jnp.int32),
        pltpu.VMEM((_CH, _D), jnp.float32),
        pltpu.VMEM((_RCH, _D), jnp.float32),
        pltpu.VMEM_SHARED((_NR, _D), jnp.float32),
        pltpu.SemaphoreType.DMA,
    ],
)


def _dinv(deg_ref):
  deg = deg_ref[:, 0:1] + deg_ref[:, 1:2] + 1.0
  return lax.rsqrt(jnp.maximum(deg, 1.0))


def _pre_body(deg_ref, x_ref, w_ref, p_ref):
  g = jnp.dot(x_ref[...], w_ref[...], preferred_element_type=jnp.float32)
  p_ref[...] = g * _dinv(deg_ref)


def _mid_body(deg_ref, s_ref, p_ref, b_ref, w_ref, o_ref):
  dinv = _dinv(deg_ref)
  agg = (s_ref[0, :, :] + s_ref[1, :, :] + p_ref[...]) * dinv
  h = jnp.maximum(agg + b_ref[...], 0.0)
  o_ref[...] = jnp.dot(h, w_ref[...], preferred_element_type=jnp.float32) * dinv


def _post_body(deg_ref, s_ref, p_ref, b_ref, w_ref, bc_ref, o_ref):
  dinv = _dinv(deg_ref)
  h = (s_ref[0, :, :] + s_ref[1, :, :] + p_ref[...]) * dinv + b_ref[...]
  o_ref[...] = (
      jnp.dot(h, w_ref[...], preferred_element_type=jnp.float32) + bc_ref[...])


def _row_specs(n_rows_feats):
  return [pl.BlockSpec((_BR, _NC), lambda i: (i, 0))] + [
      pl.BlockSpec((_BR, f), lambda i: (i, 0)) for f in n_rows_feats]


def _tc_pre(deg2, x, w1):
  grid = (_N // _BR,)
  return pl.pallas_call(
      _pre_body,
      grid=grid,
      in_specs=[
          pl.BlockSpec((_BR, _NC), lambda i: (i, 0)),
          pl.BlockSpec((_BR, _D), lambda i: (i, 0)),
          pl.BlockSpec((_D, _D), lambda i: (0, 0)),
      ],
      out_specs=pl.BlockSpec((_BR, _D), lambda i: (i, 0)),
      out_shape=jax.ShapeDtypeStruct((_N, _D), jnp.float32),
  )(deg2, x, w1)


def _tc_mid(deg2, s, p, b, w2):
  grid = (_N // _BR,)
  return pl.pallas_call(
      _mid_body,
      grid=grid,
      in_specs=[
          pl.BlockSpec((_BR, _NC), lambda i: (i, 0)),
          pl.BlockSpec((_NC, _BR, _D), lambda i: (0, i, 0)),
          pl.BlockSpec((_BR, _D), lambda i: (i, 0)),
          pl.BlockSpec((1, _D), lambda i: (0, 0)),
          pl.BlockSpec((_D, _D), lambda i: (0, 0)),
      ],
      out_specs=pl.BlockSpec((_BR, _D), lambda i: (i, 0)),
      out_shape=jax.ShapeDtypeStruct((_N, _D), jnp.float32),
  )(deg2, s, p, b, w2)


def _tc_post(deg2, s, p, b, wc, bc, out_w):
  grid = (_N // _BR,)
  return pl.pallas_call(
      _post_body,
      grid=grid,
      in_specs=[
          pl.BlockSpec((_BR, _NC), lambda i: (i, 0)),
          pl.BlockSpec((_NC, _BR, _D), lambda i: (0, i, 0)),
          pl.BlockSpec((_BR, _D), lambda i: (i, 0)),
          pl.BlockSpec((1, _D), lambda i: (0, 0)),
          pl.BlockSpec((_D, out_w), lambda i: (0, 0)),
          pl.BlockSpec((1, out_w), lambda i: (0, 0)),
      ],
      out_specs=pl.BlockSpec((_BR, out_w), lambda i: (i, 0)),
      out_shape=jax.ShapeDtypeStruct((_N, out_w), jnp.float32),
  )(deg2, s, p, b, wc, bc)


def kernel(x, edge_index, W1, b1, W2, b2, Wc, bc):
  src = edge_index[0]
  dst = edge_index[1]

  deg_parts = _hist_call(_hist_body)(dst)
  deg2 = jnp.transpose(deg_parts)

  p1 = _tc_pre(deg2, x, W1)
  s1 = _agg_call(_agg_body)(p1, src, dst)
  p2 = _tc_mid(deg2, s1, p1, b1.reshape(1, _D), W2)
  s2 = _agg_call(_agg_body)(p2, src, dst)
  out = _tc_post(deg2, s2, p2, b2.reshape(1, _D), Wc,
                 bc.reshape(1, -1), Wc.shape[1])
  return out
